# baseline (speedup 1.0000x reference)
import functools

import jax
import jax.numpy as jnp
from jax import lax
from jax.experimental import pallas as pl
from jax.experimental.pallas import tpu as pltpu
from jax.experimental.pallas import tpu_sc as plsc

_EPS = 1e-08
_NUM_CORES = 2
_NUM_SUBCORES = 16
_NW = _NUM_CORES * _NUM_SUBCORES
_NSUB = 4
_W = 8


def _rsqrt(s):
    i = lax.bitcast_convert_type(s, jnp.int32)
    i = jnp.int32(0x5F3759DF) - lax.shift_right_logical(i, 1)
    y = lax.bitcast_convert_type(i, jnp.float32)
    for _ in range(2):
        y = y * (jnp.float32(1.5) - jnp.float32(0.5) * s * y * y)
    return y


@functools.cache
def _build_kernel(n_atoms, nb_pad):
    chunk = nb_pad // _NW
    sub = chunk // _NSUB
    sub2 = 2 * sub
    groups = sub // 16

    mesh = plsc.VectorSubcoreMesh(
        core_axis_name="c",
        subcore_axis_name="s",
        num_cores=_NUM_CORES,
        num_subcores=_NUM_SUBCORES,
    )

    row_ty = pltpu.VMEM((sub2, _W), jnp.float32)

    @functools.partial(
        pl.kernel,
        out_type=jax.ShapeDtypeStruct((_NW * 16,), jnp.float32),
        mesh=mesh,
        compiler_params=pltpu.CompilerParams(
            needs_layout_passes=False, use_tc_tiling_on_sc=False),
        scratch_types=[
            pltpu.VMEM((_NSUB, sub2), jnp.int32),
            row_ty, row_ty,
            row_ty, row_ty,
            pltpu.VMEM((16,), jnp.float32),
            pltpu.SemaphoreType.DMA,
            pltpu.SemaphoreType.DMA,
        ],
    )
    def bond_loss(xp_hbm, xg_hbm, bflat_hbm, out_hbm,
                  bidx, rp0, rp1, rg0, rg1, acc_v, sem0, sem1):
        wid = lax.axis_index("s") * _NUM_CORES + lax.axis_index("c")
        base = wid * (2 * chunk)
        for p in range(_NSUB):
            pltpu.sync_copy(
                bflat_hbm.at[pl.ds(base + p * sub2, sub2)], bidx.at[p])

        rows_p = (rp0, rp1)
        rows_g = (rg0, rg1)
        sems = (sem0, sem1)

        def start(piece):
            buf = piece % 2
            return (
                pltpu.async_copy(
                    xp_hbm.at[bidx.at[piece]], rows_p[buf], sems[buf]),
                pltpu.async_copy(
                    xg_hbm.at[bidx.at[piece]], rows_g[buf], sems[buf]),
            )

        lanes2 = jax.lax.iota(jnp.int32, 16) * 2

        def accumulate(piece, acc_in):
            rp = rows_p[piece % 2]
            rg = rows_g[piece % 2]

            def body(g, acc):
                rs = g * 32 + lanes2
                re = rs + 1

                def col(rows, r, c):
                    return plsc.load_gather(
                        rows, [r, jnp.full((16,), c, jnp.int32)])

                dx = col(rp, rs, 0) - col(rp, re, 0)
                dy = col(rp, rs, 1) - col(rp, re, 1)
                dz = col(rp, rs, 2) - col(rp, re, 2)
                gx = col(rg, rs, 0) - col(rg, re, 0)
                gy = col(rg, rs, 1) - col(rg, re, 1)
                gz = col(rg, rs, 2) - col(rg, re, 2)
                sp = dx * dx + dy * dy + dz * dz + jnp.float32(_EPS)
                sg = gx * gx + gy * gy + gz * gz + jnp.float32(_EPS)
                d = sp * _rsqrt(sp) - sg * _rsqrt(sg)
                return acc + d * d

            return lax.fori_loop(0, groups, body, acc_in)

        acc = jnp.zeros((16,), jnp.float32)
        inflight = start(0)
        for piece in range(_NSUB):
            for cp in inflight:
                cp.wait()
            if piece + 1 < _NSUB:
                inflight = start(piece + 1)
            acc = accumulate(piece, acc)

        acc_v[...] = acc
        pltpu.sync_copy(acc_v, out_hbm.at[pl.ds(wid * 16, 16)])

    return bond_loss


@jax.jit
def kernel(x_pred, x_gt, bonds):
    n_atoms = x_pred.shape[0]
    nb = bonds.shape[0]
    quantum = _NW * _NSUB * 16
    nb_pad = ((nb + quantum - 1) // quantum) * quantum

    xp8 = jnp.pad(x_pred.astype(jnp.float32), ((0, 0), (0, _W - 3)))
    xg8 = jnp.pad(x_gt.astype(jnp.float32), ((0, 0), (0, _W - 3)))

    bflat = jnp.concatenate([
        bonds.astype(jnp.int32).reshape(-1),
        jnp.zeros((2 * (nb_pad - nb),), jnp.int32),
    ])

    parts = _build_kernel(n_atoms, nb_pad)(xp8, xg8, bflat)
    return jnp.sum(parts) / jnp.float32(nb)
